# Initial kernel scaffold; baseline (speedup 1.0000x reference)
import jax
import jax.numpy as jnp
from jax.experimental import pallas as pl


def kernel(h, edge_index, w, W1, b1, W2, b2):
    raise NotImplementedError("write your pallas kernel here")


# speedup vs baseline: 3.3109x; 3.3109x over previous
import jax
import jax.numpy as jnp
from jax import lax
from jax.experimental import pallas as pl
from jax.experimental.pallas import tpu as pltpu
from jax.experimental.pallas import tpu_sc as plsc

NN = 10000
NE = 320000
F = 128
WID = 144
NC = 2
NS = 16
NW = NC * NS
EPW = 10240
NEP = NW * EPW
K = 64
NB = EPW // K
NBUF = 4
EBUF = 8
ROWS = 10048
ZNS = 8
ZSTRIPE = ROWS // ZNS


def _splat(x):
    return lax.broadcast(x, (16,))


def _tc_pre(h_ref, w1t_ref, b1_ref, w2t_ref, b2_ref, u_ref, p_ref):
    hb = h_ref[...]
    u = jax.lax.dot_general(hb, w1t_ref[...], (((1,), (0,)), ((), ())),
                            precision=lax.Precision.HIGHEST,
                            preferred_element_type=jnp.float32)
    u_ref[:, :F] = u + b1_ref[...]
    col = lax.broadcasted_iota(jnp.int32, (hb.shape[0], WID - F), 1)
    u_ref[:, F:] = jnp.where(col == 0, 1.0, 0.0).astype(jnp.float32)
    p = jax.lax.dot_general(hb, w2t_ref[...], (((1,), (0,)), ((), ())),
                            precision=lax.Precision.HIGHEST,
                            preferred_element_type=jnp.float32)
    p_ref[...] = p + b2_ref[...]


def _tc_post(a_ref, b_ref, p_ref, w2bt_ref, o_ref):
    s = a_ref[...] + b_ref[...]
    cnt = jnp.maximum(s[:, F:F + 1], 1.0)
    h_n = s[:, :F] / cnt
    acc = jax.lax.dot_general(h_n, w2bt_ref[...], (((1,), (0,)), ((), ())),
                              precision=lax.Precision.HIGHEST,
                              preferred_element_type=jnp.float32)
    o_ref[...] = jnp.maximum(p_ref[...] + acc, 0.0)


def _sc_edge(u_hbm, edata_hbm, w1w_hbm, acc_hbm,
             ebuf, w1wv, gbuf, acc_s, esem, gsem, ssem):
    cid = lax.axis_index("c")
    sid = lax.axis_index("s")
    wid = cid * NS + sid

    pltpu.sync_copy(w1w_hbm, w1wv)

    @pl.loop(0, K)
    def _zrow(r):
        for j in range(WID // 16):
            gbuf[0, r, pl.ds(16 * j, 16)] = jnp.zeros((16,), jnp.float32)

    @pl.when(sid < ZNS)
    def _zero_acc():
        base = pl.multiple_of(sid * ZSTRIPE, 8)
        for i in range(ZSTRIPE // K):
            pltpu.sync_copy(gbuf.at[0], acc_s.at[pl.ds(base + i * K, K)])
        rem = ZSTRIPE % K
        if rem:
            pltpu.sync_copy(gbuf.at[0, pl.ds(0, rem)],
                            acc_s.at[pl.ds(base + (ZSTRIPE // K) * K, rem)])

    plsc.subcore_barrier()

    cs = [[w1wv[ci, pl.ds(16 * j, 16)] for j in range(F // 16)]
          for ci in range(3)]

    def edesc(b, se):
        return pltpu.make_async_copy(
            edata_hbm.at[wid, b], ebuf.at[se], esem.at[se])

    def gdesc(se, sg):
        return pltpu.make_async_copy(
            u_hbm.at[ebuf.at[se, 0]], gbuf.at[sg], gsem.at[sg])

    def sdesc(se, sg):
        return pltpu.make_async_copy(
            gbuf.at[sg], acc_s.at[ebuf.at[se, 1]], ssem.at[sg])

    edesc(0, 0).start()
    edesc(1, 1).start()
    edesc(2, 2).start()
    edesc(0, 0).wait()
    gdesc(0, 0).start()
    edesc(1, 1).wait()
    gdesc(1, 1).start()

    @pl.loop(0, NB, step=EBUF)
    def _outer(b0):
        for kk in range(EBUF):
            b = b0 + kk
            se = kk
            sg = kk % NBUF
            sgn = (kk + 2) % NBUF

            if kk >= 2:
                sdesc((kk - 2) % EBUF, sgn).wait()
            else:
                @pl.when(b >= 2)
                def _():
                    sdesc((kk - 2) % EBUF, sgn).wait()

            if kk < 5:
                edesc(b + 3, (kk + 3) % EBUF).start()
            else:
                @pl.when(b + 3 < NB)
                def _():
                    edesc(b + 3, (kk + 3) % EBUF).start()

            if kk < 6:
                edesc(b + 2, (kk + 2) % EBUF).wait()
                gdesc((kk + 2) % EBUF, sgn).start()
            else:
                @pl.when(b + 2 < NB)
                def _():
                    edesc(b + 2, (kk + 2) % EBUF).wait()
                    gdesc((kk + 2) % EBUF, sgn).start()

            gdesc(se, sg).wait()

            @plsc.parallel_loop(0, K, unroll=2)
            def _edge(e):
                ew = [plsc.bitcast(
                    plsc.load_gather(ebuf, [_splat(se), _splat(2 + ci),
                                            _splat(e)]), jnp.float32)
                    for ci in range(3)]
                for j in range(F // 16):
                    t = gbuf[sg, e, pl.ds(16 * j, 16)]
                    t = t + ew[0] * cs[0][j] + ew[1] * cs[1][j] \
                        + ew[2] * cs[2][j]
                    t = jnp.maximum(t, t * 0.01)
                    gbuf[sg, e, pl.ds(16 * j, 16)] = t

            sdesc(se, sg).start(add=True)

    sdesc((NB - 2) % EBUF, (NB - 2) % NBUF).wait()
    sdesc((NB - 1) % EBUF, (NB - 1) % NBUF).wait()
    plsc.subcore_barrier()

    @pl.when(sid < ZNS)
    def _writeout():
        base = pl.multiple_of(sid * ZSTRIPE, 8)
        pltpu.sync_copy(acc_s.at[pl.ds(base, ZSTRIPE)],
                        acc_hbm.at[cid, pl.ds(base, ZSTRIPE)])


@jax.jit
def kernel(h, edge_index, w, W1, b1, W2, b2):
    src = edge_index[0].astype(jnp.int32)
    dst = edge_index[1].astype(jnp.int32)
    pad = NEP - NE
    srcp = jnp.concatenate([src, jnp.zeros((pad,), jnp.int32)])
    dstp = jnp.concatenate([dst, jnp.full((pad,), NN, jnp.int32)])
    wp = jnp.concatenate([w.astype(jnp.float32),
                          jnp.zeros((pad, 3), jnp.float32)], axis=0)
    src4 = srcp.reshape(NW, NB, 1, K)
    dst4 = dstp.reshape(NW, NB, 1, K)
    wbits = jax.lax.bitcast_convert_type(wp, jnp.int32)
    w4 = wbits.reshape(NW, NB, K, 3).transpose(0, 1, 3, 2)
    edata = jnp.concatenate([src4, dst4, w4], axis=2)
    w1w = W1[:, F:].T.astype(jnp.float32)

    blk = 1000
    grid = NN // blk
    u_pad, p = pl.pallas_call(
        _tc_pre,
        grid=(grid,),
        in_specs=[
            pl.BlockSpec((blk, F), lambda i: (i, 0)),
            pl.BlockSpec((F, F), lambda i: (0, 0)),
            pl.BlockSpec((1, F), lambda i: (0, 0)),
            pl.BlockSpec((F, F), lambda i: (0, 0)),
            pl.BlockSpec((1, F), lambda i: (0, 0)),
        ],
        out_specs=[
            pl.BlockSpec((blk, WID), lambda i: (i, 0)),
            pl.BlockSpec((blk, F), lambda i: (i, 0)),
        ],
        out_shape=[
            jax.ShapeDtypeStruct((NN, WID), jnp.float32),
            jax.ShapeDtypeStruct((NN, F), jnp.float32),
        ],
    )(h, W1[:, :F].T, b1.reshape(1, F), W2[:, :F].T, b2.reshape(1, F))

    mesh = plsc.VectorSubcoreMesh(core_axis_name="c", subcore_axis_name="s")
    acc = pl.kernel(
        _sc_edge,
        out_type=jax.ShapeDtypeStruct((NC, ROWS, WID), jnp.float32),
        mesh=mesh,
        compiler_params=pltpu.CompilerParams(use_tc_tiling_on_sc=False,
                                             needs_layout_passes=False),
        scratch_types=[
            pltpu.VMEM((EBUF, 5, K), jnp.int32),
            pltpu.VMEM((3, F), jnp.float32),
            pltpu.VMEM((NBUF, K, WID), jnp.float32),
            pltpu.VMEM_SHARED((ROWS, WID), jnp.float32),
            pltpu.SemaphoreType.DMA((EBUF,)),
            pltpu.SemaphoreType.DMA((NBUF,)),
            pltpu.SemaphoreType.DMA((NBUF,)),
        ],
    )(u_pad, edata, w1w)

    out = pl.pallas_call(
        _tc_post,
        grid=(grid,),
        in_specs=[
            pl.BlockSpec((blk, WID), lambda i: (i, 0)),
            pl.BlockSpec((blk, WID), lambda i: (i, 0)),
            pl.BlockSpec((blk, F), lambda i: (i, 0)),
            pl.BlockSpec((F, F), lambda i: (0, 0)),
        ],
        out_specs=pl.BlockSpec((blk, F), lambda i: (i, 0)),
        out_shape=jax.ShapeDtypeStruct((NN, F), jnp.float32),
    )(acc[0, :NN], acc[1, :NN], p, W2[:, F:].T)
    return out


# speedup vs baseline: 3.5282x; 1.0657x over previous
import jax
import jax.numpy as jnp
from jax import lax
from jax.experimental import pallas as pl
from jax.experimental.pallas import tpu as pltpu
from jax.experimental.pallas import tpu_sc as plsc

NN = 10000
NE = 320000
F = 128
FH = 64
WIDA = 72
NC = 2
NS = 16
EPW = 20480
NEP = NS * EPW
K = 64
NB = EPW // K
GBUF = 4
EBUF = 8
ROWS = 10048
ZNS = 8
ZSTRIPE = ROWS // ZNS


def _splat(x):
    return lax.broadcast(x, (16,))


def _tc_pre(h_ref, w1t_ref, b1_ref, w2t_ref, b2_ref, u_ref, p_ref):
    hb = h_ref[...]
    u = jax.lax.dot_general(hb, w1t_ref[...], (((1,), (0,)), ((), ())),
                            precision=lax.Precision.HIGHEST,
                            preferred_element_type=jnp.float32)
    u = u + b1_ref[...]
    u_ref[0] = u[:, :FH]
    u_ref[1] = u[:, FH:]
    p = jax.lax.dot_general(hb, w2t_ref[...], (((1,), (0,)), ((), ())),
                            precision=lax.Precision.HIGHEST,
                            preferred_element_type=jnp.float32)
    p_ref[...] = p + b2_ref[...]


def _tc_post(a_ref, b_ref, p_ref, w2bt_ref, o_ref):
    cnt = jnp.maximum(a_ref[:, FH:FH + 1], 1.0)
    h_n = jnp.concatenate([a_ref[:, :FH], b_ref[:, :FH]], axis=1) / cnt
    acc = jax.lax.dot_general(h_n, w2bt_ref[...], (((1,), (0,)), ((), ())),
                              precision=lax.Precision.HIGHEST,
                              preferred_element_type=jnp.float32)
    o_ref[...] = jnp.maximum(p_ref[...] + acc, 0.0)


def _sc_edge(u_hbm, sd_hbm, w_hbm, w1w_hbm, acc_hbm,
             sdbuf, wbuf, w1wv, gbuf, mbuf, acc_s, u_s, esem, gsem, ssem):
    cid = lax.axis_index("c")
    sid = lax.axis_index("s")

    pltpu.sync_copy(w1w_hbm.at[cid], w1wv)

    @pl.when(sid < 5)
    def _stage_u():
        ub = pl.multiple_of(sid * 2000, 8)
        pltpu.sync_copy(u_hbm.at[cid, pl.ds(ub, 2000)], u_s.at[pl.ds(ub, 2000)])

    zv = jnp.zeros((16,), jnp.float32)

    @pl.loop(0, K)
    def _zrow(r):
        for j in range(WIDA // 16):
            mbuf[0, r, pl.ds(16 * j, 16)] = zv
        mbuf[0, r, pl.ds(WIDA - 16, 16)] = zv

    @pl.when(sid < ZNS)
    def _zero_acc():
        base = pl.multiple_of(sid * ZSTRIPE, 8)
        for i in range(ZSTRIPE // K):
            pltpu.sync_copy(mbuf.at[0], acc_s.at[pl.ds(base + i * K, K)])
        rem = ZSTRIPE % K
        if rem:
            pltpu.sync_copy(mbuf.at[0, pl.ds(0, rem)],
                            acc_s.at[pl.ds(base + (ZSTRIPE // K) * K, rem)])

    lane = lax.broadcasted_iota(jnp.int32, (16,), 0)
    cvec = jnp.where(lane == FH - (WIDA - 16), 1.0, 0.0)

    @pl.loop(0, K)
    def _crow(r):
        for sl in range(GBUF):
            mbuf[sl, r, pl.ds(WIDA - 16, 16)] = cvec

    plsc.subcore_barrier()

    cs = [[w1wv[ci, pl.ds(16 * j, 16)] for j in range(FH // 16)]
          for ci in range(3)]

    def sddesc(b, se):
        return pltpu.make_async_copy(
            sd_hbm.at[sid, b], sdbuf.at[se], esem.at[se])

    def wdesc(b, se):
        return pltpu.make_async_copy(
            w_hbm.at[sid, b], wbuf.at[se], esem.at[se])

    def gdesc(se, sg):
        return pltpu.make_async_copy(
            u_s.at[sdbuf.at[se, 0]], gbuf.at[sg], gsem.at[sg])

    def sdesc(se, sm):
        return pltpu.make_async_copy(
            mbuf.at[sm], acc_s.at[sdbuf.at[se, 1]], ssem.at[sm])

    def stage_start(b, se):
        sddesc(b, se).start()
        wdesc(b, se).start()

    def stage_wait(b, se):
        sddesc(b, se).wait()
        wdesc(b, se).wait()

    stage_start(0, 0)
    stage_start(1, 1)
    stage_start(2, 2)
    stage_wait(0, 0)
    gdesc(0, 0).start()
    stage_wait(1, 1)
    gdesc(1, 1).start()

    @pl.loop(0, NB, step=EBUF)
    def _outer(b0):
        for kk in range(EBUF):
            b = b0 + kk
            sm = kk % GBUF

            if kk >= 4:
                sdesc(kk - 4, sm).wait()
            else:
                @pl.when(b >= 4)
                def _():
                    sdesc(kk + 4, sm).wait()

            if kk < 5:
                stage_start(b + 3, kk + 3)
            else:
                @pl.when(b + 3 < NB)
                def _():
                    stage_start(b + 3, (kk + 3) % EBUF)

            if kk < 6:
                stage_wait(b + 2, kk + 2)
                gdesc(kk + 2, (kk + 2) % GBUF).start()
            else:
                @pl.when(b + 2 < NB)
                def _():
                    stage_wait(b + 2, (kk + 2) % EBUF)
                    gdesc((kk + 2) % EBUF, (kk + 2) % GBUF).start()

            gdesc(kk, sm).wait()

            @plsc.parallel_loop(0, K, unroll=2)
            def _edge(e):
                se_v = _splat(kk)
                e_v = _splat(e)
                ws = [plsc.load_gather(wbuf, [se_v, e_v, _splat(ci)])
                      for ci in range(3)]
                for j in range(FH // 16):
                    t = gbuf[sm, e, pl.ds(16 * j, 16)]
                    t = t + ws[0] * cs[0][j] + ws[1] * cs[1][j] \
                        + ws[2] * cs[2][j]
                    t = jnp.maximum(t, t * 0.01)
                    mbuf[sm, e, pl.ds(16 * j, 16)] = t

            sdesc(kk, sm).start(add=True)

    for b in range(NB - 4, NB):
        sdesc(b % EBUF, b % GBUF).wait()
    plsc.subcore_barrier()

    @pl.when(sid < ZNS)
    def _writeout():
        base = pl.multiple_of(sid * ZSTRIPE, 8)
        pltpu.sync_copy(acc_s.at[pl.ds(base, ZSTRIPE)],
                        acc_hbm.at[cid, pl.ds(base, ZSTRIPE)])


@jax.jit
def kernel(h, edge_index, w, W1, b1, W2, b2):
    src = edge_index[0].astype(jnp.int32)
    dst = edge_index[1].astype(jnp.int32)
    pad = NEP - NE
    srcp = jnp.concatenate([src, jnp.zeros((pad,), jnp.int32)])
    dstp = jnp.concatenate([dst, jnp.full((pad,), NN, jnp.int32)])
    wp = jnp.concatenate([w.astype(jnp.float32),
                          jnp.zeros((pad, 3), jnp.float32)], axis=0)
    sd = jnp.stack([srcp.reshape(NS, NB, K),
                    dstp.reshape(NS, NB, K)], axis=2)
    w_r = wp.reshape(NS, NB, K, 3)
    w1w = W1[:, F:].T.astype(jnp.float32)
    w1w_halves = jnp.stack([w1w[:, :FH], w1w[:, FH:]])

    blk = 1000
    grid = NN // blk
    u_pad, p = pl.pallas_call(
        _tc_pre,
        grid=(grid,),
        in_specs=[
            pl.BlockSpec((blk, F), lambda i: (i, 0)),
            pl.BlockSpec((F, F), lambda i: (0, 0)),
            pl.BlockSpec((1, F), lambda i: (0, 0)),
            pl.BlockSpec((F, F), lambda i: (0, 0)),
            pl.BlockSpec((1, F), lambda i: (0, 0)),
        ],
        out_specs=[
            pl.BlockSpec((NC, blk, FH), lambda i: (0, i, 0)),
            pl.BlockSpec((blk, F), lambda i: (i, 0)),
        ],
        out_shape=[
            jax.ShapeDtypeStruct((NC, NN, FH), jnp.float32),
            jax.ShapeDtypeStruct((NN, F), jnp.float32),
        ],
    )(h, W1[:, :F].T, b1.reshape(1, F), W2[:, :F].T, b2.reshape(1, F))

    mesh = plsc.VectorSubcoreMesh(core_axis_name="c", subcore_axis_name="s")
    acc = pl.kernel(
        _sc_edge,
        out_type=jax.ShapeDtypeStruct((NC, ROWS, WIDA), jnp.float32),
        mesh=mesh,
        compiler_params=pltpu.CompilerParams(use_tc_tiling_on_sc=False,
                                             needs_layout_passes=False),
        scratch_types=[
            pltpu.VMEM((EBUF, 2, K), jnp.int32),
            pltpu.VMEM((EBUF, K, 3), jnp.float32),
            pltpu.VMEM((3, FH), jnp.float32),
            pltpu.VMEM((GBUF, K, FH), jnp.float32),
            pltpu.VMEM((GBUF, K, WIDA), jnp.float32),
            pltpu.VMEM_SHARED((ROWS, WIDA), jnp.float32),
            pltpu.VMEM_SHARED((NN, FH), jnp.float32),
            pltpu.SemaphoreType.DMA((EBUF,)),
            pltpu.SemaphoreType.DMA((GBUF,)),
            pltpu.SemaphoreType.DMA((GBUF,)),
        ],
    )(u_pad, sd, w_r, w1w_halves)

    out = pl.pallas_call(
        _tc_post,
        grid=(grid,),
        in_specs=[
            pl.BlockSpec((blk, WIDA), lambda i: (i, 0)),
            pl.BlockSpec((blk, WIDA), lambda i: (i, 0)),
            pl.BlockSpec((blk, F), lambda i: (i, 0)),
            pl.BlockSpec((F, F), lambda i: (0, 0)),
        ],
        out_specs=pl.BlockSpec((blk, F), lambda i: (i, 0)),
        out_shape=jax.ShapeDtypeStruct((NN, F), jnp.float32),
    )(acc[0, :NN], acc[1, :NN], p, W2[:, F:].T)
    return out


# speedup vs baseline: 3.5692x; 1.0116x over previous
import jax
import jax.numpy as jnp
from jax import lax
from jax.experimental import pallas as pl
from jax.experimental.pallas import tpu as pltpu
from jax.experimental.pallas import tpu_sc as plsc

NN = 10000
NE = 320000
F = 128
FH = 64
WIDA = 72
NC = 2
NS = 16
EPW = 20480
NEP = NS * EPW
K = 64
NB = EPW // K
GBUF = 4
CH = 8
NCH = NB // CH
ROWS = 10048
ZNS = 8
ZSTRIPE = ROWS // ZNS


def _splat(x):
    return lax.broadcast(x, (16,))


def _tc_pre(h_ref, w1t_ref, b1_ref, w2t_ref, b2_ref, u_ref, p_ref):
    hb = h_ref[...]
    u = jax.lax.dot_general(hb, w1t_ref[...], (((1,), (0,)), ((), ())),
                            precision=lax.Precision.HIGHEST,
                            preferred_element_type=jnp.float32)
    u = u + b1_ref[...]
    u_ref[0] = u[:, :FH]
    u_ref[1] = u[:, FH:]
    p = jax.lax.dot_general(hb, w2t_ref[...], (((1,), (0,)), ((), ())),
                            precision=lax.Precision.HIGHEST,
                            preferred_element_type=jnp.float32)
    p_ref[...] = p + b2_ref[...]


def _tc_post(a_ref, b_ref, p_ref, w2bt_ref, o_ref):
    cnt = jnp.maximum(a_ref[:, FH:FH + 1], 1.0)
    h_n = jnp.concatenate([a_ref[:, :FH], b_ref[:, :FH]], axis=1) / cnt
    acc = jax.lax.dot_general(h_n, w2bt_ref[...], (((1,), (0,)), ((), ())),
                              precision=lax.Precision.HIGHEST,
                              preferred_element_type=jnp.float32)
    o_ref[...] = jnp.maximum(p_ref[...] + acc, 0.0)


def _sc_edge(u_hbm, sd_hbm, w_hbm, w1w_hbm, acc_hbm,
             sdbuf, wbuf, w1wv, gbuf, mbuf, acc_s, u_s, csem, gsem, ssem):
    cid = lax.axis_index("c")
    sid = lax.axis_index("s")

    pltpu.sync_copy(w1w_hbm.at[cid], w1wv)

    @pl.when(sid < 5)
    def _stage_u():
        ub = pl.multiple_of(sid * 2000, 8)
        pltpu.sync_copy(u_hbm.at[cid, pl.ds(ub, 2000)], u_s.at[pl.ds(ub, 2000)])

    zv = jnp.zeros((16,), jnp.float32)

    @pl.loop(0, K)
    def _zrow(r):
        for j in range(WIDA // 16):
            mbuf[0, r, pl.ds(16 * j, 16)] = zv
        mbuf[0, r, pl.ds(WIDA - 16, 16)] = zv

    @pl.when(sid < ZNS)
    def _zero_acc():
        base = pl.multiple_of(sid * ZSTRIPE, 8)
        for i in range(ZSTRIPE // K):
            pltpu.sync_copy(mbuf.at[0], acc_s.at[pl.ds(base + i * K, K)])
        rem = ZSTRIPE % K
        if rem:
            pltpu.sync_copy(mbuf.at[0, pl.ds(0, rem)],
                            acc_s.at[pl.ds(base + (ZSTRIPE // K) * K, rem)])

    lane = lax.broadcasted_iota(jnp.int32, (16,), 0)
    cvec = jnp.where(lane == FH - (WIDA - 16), 1.0, 0.0)

    @pl.loop(0, K)
    def _crow(r):
        for sl in range(GBUF):
            mbuf[sl, r, pl.ds(WIDA - 16, 16)] = cvec

    plsc.subcore_barrier()

    cs = [[w1wv[ci, pl.ds(16 * j, 16)] for j in range(FH // 16)]
          for ci in range(3)]

    def cdesc(c, sl):
        return (pltpu.make_async_copy(sd_hbm.at[sid, c], sdbuf.at[sl],
                                      csem.at[sl]),
                pltpu.make_async_copy(w_hbm.at[sid, c], wbuf.at[sl],
                                      csem.at[sl]))

    def gdesc(hh, j, sg):
        return pltpu.make_async_copy(
            u_s.at[sdbuf.at[hh, 0, j]], gbuf.at[sg], gsem.at[sg])

    def sdesc(hh, j, sm):
        return pltpu.make_async_copy(
            mbuf.at[sm], acc_s.at[sdbuf.at[hh, 1, j]], ssem.at[sm])

    for d in cdesc(0, 0):
        d.start()
    for d in cdesc(0, 0):
        d.wait()
    gdesc(0, 0, 0).start()
    gdesc(0, 1, 1).start()

    @pl.loop(0, NB, step=2 * CH)
    def _outer(b0):
        c0 = b0 // CH
        for kk in range(2 * CH):
            b = b0 + kk
            h = kk // CH
            j = kk % CH
            sm = kk % GBUF

            hb4 = ((kk - 4) % (2 * CH)) // CH
            jb4 = (kk - 4) % CH
            if kk >= 4:
                sdesc(hb4, jb4, sm).wait()
            else:
                @pl.when(b >= 4)
                def _():
                    sdesc(hb4, jb4, sm).wait()

            if kk == 4:
                for d in cdesc(c0 + 1, 1):
                    d.start()
            if kk == CH + 4:
                @pl.when(b0 + 2 * CH < NB)
                def _():
                    for d in cdesc(c0 + 2, 0):
                        d.start()

            j2 = (kk + 2) % CH
            h2 = ((kk + 2) % (2 * CH)) // CH
            sg2 = (kk + 2) % GBUF
            if kk == CH - 2:
                for d in cdesc(c0 + 1, 1):
                    d.wait()
                gdesc(h2, j2, sg2).start()
            elif kk == 2 * CH - 2:
                @pl.when(b + 2 < NB)
                def _():
                    for d in cdesc(c0 + 2, 0):
                        d.wait()
                    gdesc(h2, j2, sg2).start()
            elif kk == 2 * CH - 1:
                @pl.when(b + 2 < NB)
                def _():
                    gdesc(h2, j2, sg2).start()
            else:
                gdesc(h2, j2, sg2).start()

            gdesc(h, j, sm).wait()

            @plsc.parallel_loop(0, K, unroll=2)
            def _edge(e):
                h_v = _splat(h)
                j_v = _splat(j)
                e_v = _splat(e)
                ws = [plsc.load_gather(wbuf, [h_v, j_v, e_v, _splat(ci)])
                      for ci in range(3)]
                for g in range(FH // 16):
                    t = gbuf[sm, e, pl.ds(16 * g, 16)]
                    t = t + ws[0] * cs[0][g] + ws[1] * cs[1][g] \
                        + ws[2] * cs[2][g]
                    t = jnp.maximum(t, t * 0.01)
                    mbuf[sm, e, pl.ds(16 * g, 16)] = t

            sdesc(h, j, sm).start(add=True)

    for kk in range(2 * CH - 4, 2 * CH):
        sdesc(kk // CH, kk % CH, kk % GBUF).wait()
    plsc.subcore_barrier()

    @pl.when(sid < ZNS)
    def _writeout():
        base = pl.multiple_of(sid * ZSTRIPE, 8)
        pltpu.sync_copy(acc_s.at[pl.ds(base, ZSTRIPE)],
                        acc_hbm.at[cid, pl.ds(base, ZSTRIPE)])


@jax.jit
def kernel(h, edge_index, w, W1, b1, W2, b2):
    src = edge_index[0].astype(jnp.int32)
    dst = edge_index[1].astype(jnp.int32)
    pad = NEP - NE
    srcp = jnp.concatenate([src, jnp.zeros((pad,), jnp.int32)])
    dstp = jnp.concatenate([dst, jnp.full((pad,), NN, jnp.int32)])
    wp = jnp.concatenate([w.astype(jnp.float32),
                          jnp.zeros((pad, 3), jnp.float32)], axis=0)
    sd = jnp.stack([srcp.reshape(NS, NCH, CH, K),
                    dstp.reshape(NS, NCH, CH, K)], axis=2)
    w_r = wp.reshape(NS, NCH, CH, K, 3)
    w1w = W1[:, F:].T.astype(jnp.float32)
    w1w_halves = jnp.stack([w1w[:, :FH], w1w[:, FH:]])

    blk = 1000
    grid = NN // blk
    u_pad, p = pl.pallas_call(
        _tc_pre,
        grid=(grid,),
        in_specs=[
            pl.BlockSpec((blk, F), lambda i: (i, 0)),
            pl.BlockSpec((F, F), lambda i: (0, 0)),
            pl.BlockSpec((1, F), lambda i: (0, 0)),
            pl.BlockSpec((F, F), lambda i: (0, 0)),
            pl.BlockSpec((1, F), lambda i: (0, 0)),
        ],
        out_specs=[
            pl.BlockSpec((NC, blk, FH), lambda i: (0, i, 0)),
            pl.BlockSpec((blk, F), lambda i: (i, 0)),
        ],
        out_shape=[
            jax.ShapeDtypeStruct((NC, NN, FH), jnp.float32),
            jax.ShapeDtypeStruct((NN, F), jnp.float32),
        ],
    )(h, W1[:, :F].T, b1.reshape(1, F), W2[:, :F].T, b2.reshape(1, F))

    mesh = plsc.VectorSubcoreMesh(core_axis_name="c", subcore_axis_name="s")
    acc = pl.kernel(
        _sc_edge,
        out_type=jax.ShapeDtypeStruct((NC, ROWS, WIDA), jnp.float32),
        mesh=mesh,
        compiler_params=pltpu.CompilerParams(use_tc_tiling_on_sc=False,
                                             needs_layout_passes=False),
        scratch_types=[
            pltpu.VMEM((2, 2, CH, K), jnp.int32),
            pltpu.VMEM((2, CH, K, 3), jnp.float32),
            pltpu.VMEM((3, FH), jnp.float32),
            pltpu.VMEM((GBUF, K, FH), jnp.float32),
            pltpu.VMEM((GBUF, K, WIDA), jnp.float32),
            pltpu.VMEM_SHARED((ROWS, WIDA), jnp.float32),
            pltpu.VMEM_SHARED((NN, FH), jnp.float32),
            pltpu.SemaphoreType.DMA((2,)),
            pltpu.SemaphoreType.DMA((GBUF,)),
            pltpu.SemaphoreType.DMA((GBUF,)),
        ],
    )(u_pad, sd, w_r, w1w_halves)

    out = pl.pallas_call(
        _tc_post,
        grid=(grid,),
        in_specs=[
            pl.BlockSpec((blk, WIDA), lambda i: (i, 0)),
            pl.BlockSpec((blk, WIDA), lambda i: (i, 0)),
            pl.BlockSpec((blk, F), lambda i: (i, 0)),
            pl.BlockSpec((F, F), lambda i: (0, 0)),
        ],
        out_specs=pl.BlockSpec((blk, F), lambda i: (i, 0)),
        out_shape=jax.ShapeDtypeStruct((NN, F), jnp.float32),
    )(acc[0, :NN], acc[1, :NN], p, W2[:, F:].T)
    return out


# speedup vs baseline: 3.9290x; 1.1008x over previous
import jax
import jax.numpy as jnp
from jax import lax
from jax.experimental import pallas as pl
from jax.experimental.pallas import tpu as pltpu
from jax.experimental.pallas import tpu_sc as plsc

NN = 10000
NE = 320000
F = 128
FH = 64
WIDA = 72
NC = 2
NS = 16
EPW = NE // NS
K = 50
NB = EPW // K
GBUF = 4
CH = 10
NCH = NB // CH
ROWS = 10048
ZNS = 8
ZSTRIPE = ROWS // ZNS


def _splat(x):
    return lax.broadcast(x, (16,))


def _tc_pre(h_ref, w1t_ref, b1_ref, w2t_ref, b2_ref, u_ref, p_ref):
    hb = h_ref[...]
    u = jax.lax.dot_general(hb, w1t_ref[...], (((1,), (0,)), ((), ())),
                            precision=lax.Precision.HIGHEST,
                            preferred_element_type=jnp.float32)
    u = u + b1_ref[...]
    u_ref[0] = u[:, :FH]
    u_ref[1] = u[:, FH:]
    p = jax.lax.dot_general(hb, w2t_ref[...], (((1,), (0,)), ((), ())),
                            precision=lax.Precision.HIGHEST,
                            preferred_element_type=jnp.float32)
    p_ref[...] = p + b2_ref[...]


def _tc_post(a_ref, p_ref, w2bt_ref, o_ref):
    cnt = jnp.maximum(a_ref[0, :, FH:FH + 1], 1.0)
    h_n = jnp.concatenate([a_ref[0, :, :FH], a_ref[1, :, :FH]], axis=1) / cnt
    acc = jax.lax.dot_general(h_n, w2bt_ref[...], (((1,), (0,)), ((), ())),
                              precision=lax.Precision.HIGHEST,
                              preferred_element_type=jnp.float32)
    o_ref[...] = jnp.maximum(p_ref[...] + acc, 0.0)


def _sc_edge(u_hbm, src_hbm, dst_hbm, w_hbm, w1w_hbm, acc_hbm,
             sbuf, dbuf, wbuf, w1wv, gbuf, mbuf, acc_s, u_s,
             csem, gsem, ssem):
    cid = lax.axis_index("c")
    sid = lax.axis_index("s")

    pltpu.sync_copy(w1w_hbm.at[cid], w1wv)

    @pl.when(sid < 5)
    def _stage_u():
        ub = pl.multiple_of(sid * 2000, 8)
        pltpu.sync_copy(u_hbm.at[cid, pl.ds(ub, 2000)], u_s.at[pl.ds(ub, 2000)])

    zv = jnp.zeros((16,), jnp.float32)

    @pl.loop(0, K)
    def _zrow(r):
        for j in range(WIDA // 16):
            mbuf[0, r, pl.ds(16 * j, 16)] = zv
        mbuf[0, r, pl.ds(WIDA - 16, 16)] = zv

    @pl.when(sid < ZNS)
    def _zero_acc():
        base = pl.multiple_of(sid * ZSTRIPE, 8)
        for i in range(ZSTRIPE // K):
            pltpu.sync_copy(mbuf.at[0], acc_s.at[pl.ds(base + i * K, K)])
        rem = ZSTRIPE % K
        if rem:
            pltpu.sync_copy(mbuf.at[0, pl.ds(0, rem)],
                            acc_s.at[pl.ds(base + (ZSTRIPE // K) * K, rem)])

    lane = lax.broadcasted_iota(jnp.int32, (16,), 0)
    cvec = jnp.where(lane == FH - (WIDA - 16), 1.0, 0.0)

    @pl.loop(0, K)
    def _crow(r):
        for sl in range(GBUF):
            mbuf[sl, r, pl.ds(WIDA - 16, 16)] = cvec

    plsc.subcore_barrier()

    cs = [[w1wv[ci, pl.ds(16 * j, 16)] for j in range(FH // 16)]
          for ci in range(3)]

    def cdesc(c, sl):
        return (pltpu.make_async_copy(src_hbm.at[sid, c], sbuf.at[sl],
                                      csem.at[sl]),
                pltpu.make_async_copy(dst_hbm.at[sid, c], dbuf.at[sl],
                                      csem.at[sl]),
                pltpu.make_async_copy(w_hbm.at[sid, c], wbuf.at[sl],
                                      csem.at[sl]))

    def gdesc(hh, j, sg):
        return pltpu.make_async_copy(
            u_s.at[sbuf.at[hh, j]], gbuf.at[sg], gsem.at[sg])

    def sdesc(hh, j, sm):
        return pltpu.make_async_copy(
            mbuf.at[sm], acc_s.at[dbuf.at[hh, j]], ssem.at[sm])

    for d in cdesc(0, 0):
        d.start()
    for d in cdesc(0, 0):
        d.wait()
    gdesc(0, 0, 0).start()
    gdesc(0, 1, 1).start()

    @pl.loop(0, NB, step=2 * CH)
    def _outer(b0):
        c0 = b0 // CH
        for kk in range(2 * CH):
            b = b0 + kk
            h = kk // CH
            j = kk % CH
            sm = kk % GBUF

            hb4 = ((kk - 4) % (2 * CH)) // CH
            jb4 = (kk - 4) % CH
            if kk >= 4:
                sdesc(hb4, jb4, sm).wait()
            else:
                @pl.when(b >= 4)
                def _():
                    sdesc(hb4, jb4, sm).wait()

            if kk == 4:
                for d in cdesc(c0 + 1, 1):
                    d.start()
            if kk == CH + 4:
                @pl.when(b0 + 2 * CH < NB)
                def _():
                    for d in cdesc(c0 + 2, 0):
                        d.start()

            j2 = (kk + 2) % CH
            h2 = ((kk + 2) % (2 * CH)) // CH
            sg2 = (kk + 2) % GBUF
            if kk == CH - 2:
                for d in cdesc(c0 + 1, 1):
                    d.wait()
                gdesc(h2, j2, sg2).start()
            elif kk == 2 * CH - 2:
                @pl.when(b + 2 < NB)
                def _():
                    for d in cdesc(c0 + 2, 0):
                        d.wait()
                    gdesc(h2, j2, sg2).start()
            elif kk == 2 * CH - 1:
                @pl.when(b + 2 < NB)
                def _():
                    gdesc(h2, j2, sg2).start()
            else:
                gdesc(h2, j2, sg2).start()

            gdesc(h, j, sm).wait()

            @plsc.parallel_loop(0, K, unroll=2)
            def _edge(e):
                h_v = _splat(h)
                j_v = _splat(j)
                e_v = _splat(e)
                ws = [plsc.load_gather(wbuf, [h_v, j_v, e_v, _splat(ci)])
                      for ci in range(3)]
                for g in range(FH // 16):
                    t = gbuf[sm, e, pl.ds(16 * g, 16)]
                    t = t + ws[0] * cs[0][g] + ws[1] * cs[1][g] \
                        + ws[2] * cs[2][g]
                    t = jnp.maximum(t, t * 0.01)
                    mbuf[sm, e, pl.ds(16 * g, 16)] = t

            sdesc(h, j, sm).start(add=True)

    for kk in range(2 * CH - 4, 2 * CH):
        sdesc(kk // CH, kk % CH, kk % GBUF).wait()
    plsc.subcore_barrier()

    @pl.when(sid < ZNS)
    def _writeout():
        base = pl.multiple_of(sid * ZSTRIPE, 8)
        pltpu.sync_copy(acc_s.at[pl.ds(base, ZSTRIPE)],
                        acc_hbm.at[cid, pl.ds(base, ZSTRIPE)])


@jax.jit
def kernel(h, edge_index, w, W1, b1, W2, b2):
    src_r = edge_index[0].astype(jnp.int32).reshape(NS, NCH, CH, K)
    dst_r = edge_index[1].astype(jnp.int32).reshape(NS, NCH, CH, K)
    w_r = w.astype(jnp.float32).reshape(NS, NCH, CH, K, 3)
    w1w = W1[:, F:].T.astype(jnp.float32)
    w1w_halves = jnp.stack([w1w[:, :FH], w1w[:, FH:]])

    blk = 1000
    grid = NN // blk
    u_pad, p = pl.pallas_call(
        _tc_pre,
        grid=(grid,),
        in_specs=[
            pl.BlockSpec((blk, F), lambda i: (i, 0)),
            pl.BlockSpec((F, F), lambda i: (0, 0)),
            pl.BlockSpec((1, F), lambda i: (0, 0)),
            pl.BlockSpec((F, F), lambda i: (0, 0)),
            pl.BlockSpec((1, F), lambda i: (0, 0)),
        ],
        out_specs=[
            pl.BlockSpec((NC, blk, FH), lambda i: (0, i, 0)),
            pl.BlockSpec((blk, F), lambda i: (i, 0)),
        ],
        out_shape=[
            jax.ShapeDtypeStruct((NC, NN, FH), jnp.float32),
            jax.ShapeDtypeStruct((NN, F), jnp.float32),
        ],
    )(h, W1[:, :F].T, b1.reshape(1, F), W2[:, :F].T, b2.reshape(1, F))

    mesh = plsc.VectorSubcoreMesh(core_axis_name="c", subcore_axis_name="s")
    acc = pl.kernel(
        _sc_edge,
        out_type=jax.ShapeDtypeStruct((NC, ROWS, WIDA), jnp.float32),
        mesh=mesh,
        compiler_params=pltpu.CompilerParams(use_tc_tiling_on_sc=False,
                                             needs_layout_passes=False),
        scratch_types=[
            pltpu.VMEM((2, CH, K), jnp.int32),
            pltpu.VMEM((2, CH, K), jnp.int32),
            pltpu.VMEM((2, CH, K, 3), jnp.float32),
            pltpu.VMEM((3, FH), jnp.float32),
            pltpu.VMEM((GBUF, K, FH), jnp.float32),
            pltpu.VMEM((GBUF, K, WIDA), jnp.float32),
            pltpu.VMEM_SHARED((ROWS, WIDA), jnp.float32),
            pltpu.VMEM_SHARED((NN, FH), jnp.float32),
            pltpu.SemaphoreType.DMA((2,)),
            pltpu.SemaphoreType.DMA((GBUF,)),
            pltpu.SemaphoreType.DMA((GBUF,)),
        ],
    )(u_pad, src_r, dst_r, w_r, w1w_halves)

    out = pl.pallas_call(
        _tc_post,
        grid=(grid,),
        in_specs=[
            pl.BlockSpec((NC, blk, WIDA), lambda i: (0, i, 0)),
            pl.BlockSpec((blk, F), lambda i: (i, 0)),
            pl.BlockSpec((F, F), lambda i: (0, 0)),
        ],
        out_specs=pl.BlockSpec((blk, F), lambda i: (i, 0)),
        out_shape=jax.ShapeDtypeStruct((NN, F), jnp.float32),
    )(acc, p, W2[:, F:].T)
    return out
